# Initial kernel scaffold; baseline (speedup 1.0000x reference)
import functools

import jax
import jax.numpy as jnp
from jax import lax
from jax.experimental import pallas as pl
from jax.experimental.pallas import tpu as pltpu
from jax.experimental.pallas import tpu_sc as plsc

_N = 10000
_E = 320000
_D = 128
_DE = 16

_NC = 2
_NS = 16
_NW = _NC * _NS
_EPW = _E // _NW
_SUB = 80
_C = 2000
_NSUB = _C // _SUB
_NCH = _EPW // _C
_RPS = _N // _NS


def _tc_pre_body(x_ref, we_ref, p1_ref, p2_ref):
    x = x_ref[...]
    w = we_ref[...]
    p1_ref[...] = jnp.dot(x, w[0:_D, :], preferred_element_type=jnp.float32)
    p2_ref[...] = jnp.dot(x, w[_D:2 * _D, :], preferred_element_type=jnp.float32)


_tc_pre = pl.pallas_call(
    _tc_pre_body,
    out_shape=(
        jax.ShapeDtypeStruct((_N, _DE), jnp.float32),
        jax.ShapeDtypeStruct((_N, _DE), jnp.float32),
    ),
)


def _tc_base_body(ea_ref, w_ref, b_ref, o_ref):
    o_ref[...] = (
        jnp.dot(ea_ref[...], w_ref[...], preferred_element_type=jnp.float32)
        + b_ref[...]
    )


_E8 = _E // 8
_BLK = 4000
_tc_base = pl.pallas_call(
    _tc_base_body,
    grid=(_E8 // _BLK,),
    in_specs=[
        pl.BlockSpec((_BLK, 8 * _DE), lambda i: (i, 0)),
        pl.BlockSpec((8 * _DE, 8 * _DE), lambda i: (0, 0)),
        pl.BlockSpec((1, 8 * _DE), lambda i: (0, 0)),
    ],
    out_specs=pl.BlockSpec((_BLK, 8 * _DE), lambda i: (i, 0)),
    out_shape=jax.ShapeDtypeStruct((_E8, 8 * _DE), jnp.float32),
)


def _sc_body(p1_hbm, p2_hbm, base_hbm, si_hbm, di_hbm, eout_hbm, part_hbm,
             idx_s, idx_d, ps, pd, en, agg_sh, sem_g1, sem_g2, sem_st, sem_sc):
    c = lax.axis_index("c")
    s = lax.axis_index("s")
    wid = c * _NS + s

    zero = jnp.zeros((_DE,), jnp.float32)

    def _z(i, carry):
        ps[i] = zero
        return carry

    lax.fori_loop(0, _RPS, _z, 0)
    pltpu.sync_copy(ps.at[pl.ds(0, _RPS)], agg_sh.at[pl.ds(s * _RPS, _RPS)])
    plsc.subcore_barrier()

    def _chunk(g, carry):
        e0 = wid * _EPW + g * _C
        r0 = e0 // _SUB
        pltpu.sync_copy(si_hbm.at[pl.ds(r0, _NSUB)], idx_s)
        pltpu.sync_copy(di_hbm.at[pl.ds(r0, _NSUB)], idx_d)
        h_base = pltpu.async_copy(base_hbm.at[pl.ds(e0, _C)], en, sem_st)

        def _fire(j, cc):
            pltpu.async_copy(p1_hbm.at[idx_s.at[j]],
                             ps.at[pl.ds(j * _SUB, _SUB)], sem_g1)
            pltpu.async_copy(p2_hbm.at[idx_d.at[j]],
                             pd.at[pl.ds(j * _SUB, _SUB)], sem_g2)
            return cc

        lax.fori_loop(0, _NSUB, _fire, 0)
        h_base.wait()
        pltpu.make_async_copy(eout_hbm.at[pl.ds(0, _C)], ps, sem_g1).wait()
        pltpu.make_async_copy(eout_hbm.at[pl.ds(0, _C)], pd, sem_g2).wait()

        def _cmp(i, cc):
            en[i] = jnp.maximum(en[i] + ps[i] + pd[i], 0.0)
            return cc

        lax.fori_loop(0, _C, _cmp, 0)

        h_out = pltpu.async_copy(en, eout_hbm.at[pl.ds(e0, _C)], sem_st)

        def _scat(j, cc):
            pltpu.async_copy(en.at[pl.ds(j * _SUB, _SUB)],
                             agg_sh.at[idx_d.at[j]], sem_sc, add=True)
            return cc

        lax.fori_loop(0, _NSUB, _scat, 0)
        h_out.wait()
        pltpu.make_async_copy(eout_hbm.at[pl.ds(0, _C)], en, sem_sc).wait()
        return carry

    lax.fori_loop(0, _NCH, _chunk, 0)

    plsc.subcore_barrier()
    pltpu.sync_copy(agg_sh.at[pl.ds(s * _RPS, _RPS)], ps.at[pl.ds(0, _RPS)])
    pltpu.sync_copy(ps.at[pl.ds(0, _RPS)],
                    part_hbm.at[c, pl.ds(s * _RPS, _RPS)])


_sc_edges = functools.partial(
    pl.kernel,
    out_type=(
        jax.ShapeDtypeStruct((_E, _DE), jnp.float32),
        jax.ShapeDtypeStruct((_NC, _N, _DE), jnp.float32),
    ),
    mesh=plsc.VectorSubcoreMesh(core_axis_name="c", subcore_axis_name="s",
                                num_cores=_NC, num_subcores=_NS),
    scratch_types=[
        pltpu.VMEM((_NSUB, _SUB), jnp.int32),
        pltpu.VMEM((_NSUB, _SUB), jnp.int32),
        pltpu.VMEM((_C, _DE), jnp.float32),
        pltpu.VMEM((_C, _DE), jnp.float32),
        pltpu.VMEM((_C, _DE), jnp.float32),
        pltpu.VMEM_SHARED((_N, _DE), jnp.float32),
        pltpu.SemaphoreType.DMA,
        pltpu.SemaphoreType.DMA,
        pltpu.SemaphoreType.DMA,
        pltpu.SemaphoreType.DMA,
    ],
)(_sc_body)


def _tc_post_body(x_ref, pp_ref, wn_ref, bn_ref, o_ref):
    x = x_ref[...]
    agg = pp_ref[0] + pp_ref[1]
    wn = wn_ref[...]
    o = (
        jnp.dot(x, wn[0:_D, :], preferred_element_type=jnp.float32)
        + jnp.dot(agg, wn[_D:_D + _DE, :], preferred_element_type=jnp.float32)
        + bn_ref[...]
    )
    o_ref[...] = jnp.maximum(o, 0.0)


_tc_post = pl.pallas_call(
    _tc_post_body,
    out_shape=jax.ShapeDtypeStruct((_N, _D), jnp.float32),
)


def kernel(x, edge_index, edge_attr, W_e, b_e, W_n, b_n):
    src = edge_index[0].reshape(_E // _SUB, _SUB)
    dst = edge_index[1].reshape(_E // _SUB, _SUB)
    p1, p2 = _tc_pre(x, W_e)
    w3b = jnp.kron(jnp.eye(8, dtype=jnp.float32), W_e[2 * _D:])
    b8 = jnp.tile(b_e, 8).reshape(1, 8 * _DE)
    base = _tc_base(edge_attr.reshape(_E8, 8 * _DE), w3b, b8).reshape(_E, _DE)
    eout, part = _sc_edges(p1, p2, base, src, dst)
    x_new = _tc_post(x, part, W_n, b_n.reshape(1, _D))
    return x_new, eout


# speedup vs baseline: 6.2170x; 6.2170x over previous
import functools

import jax
import jax.numpy as jnp
from jax import lax
from jax.experimental import pallas as pl
from jax.experimental.pallas import tpu as pltpu
from jax.experimental.pallas import tpu_sc as plsc

_N = 10000
_E = 320000
_D = 128
_DE = 16

_NC = 2
_NS = 16
_NW = _NC * _NS
_EPW = _E // _NW
_SUB = 125
_C = 2000
_NSUB = _C // _SUB
_NCH = _EPW // _C
_RPS = 624
_RTL = _N - _RPS * _NS


def _tc_pre_body(x_ref, we_ref, p1_ref, p2_ref):
    x = x_ref[...]
    w = we_ref[...]
    p1_ref[...] = jnp.dot(x, w[0:_D, :], preferred_element_type=jnp.float32)
    p2_ref[...] = jnp.dot(x, w[_D:2 * _D, :], preferred_element_type=jnp.float32)


_tc_pre = pl.pallas_call(
    _tc_pre_body,
    out_shape=(
        jax.ShapeDtypeStruct((_N, _DE), jnp.float32),
        jax.ShapeDtypeStruct((_N, _DE), jnp.float32),
    ),
)


def _tc_base_body(ea_ref, w_ref, b_ref, o_ref):
    o_ref[...] = (
        jnp.dot(ea_ref[...], w_ref[...], preferred_element_type=jnp.float32)
        + b_ref[...]
    )


_E8 = _E // 8
_BLK = 4000
_tc_base = pl.pallas_call(
    _tc_base_body,
    grid=(_E8 // _BLK,),
    in_specs=[
        pl.BlockSpec((_BLK, 8 * _DE), lambda i: (i, 0)),
        pl.BlockSpec((8 * _DE, 8 * _DE), lambda i: (0, 0)),
        pl.BlockSpec((1, 8 * _DE), lambda i: (0, 0)),
    ],
    out_specs=pl.BlockSpec((_BLK, 8 * _DE), lambda i: (i, 0)),
    out_shape=jax.ShapeDtypeStruct((_E8, 8 * _DE), jnp.float32),
)


def _sc_body(p1_hbm, p2_hbm, base_hbm, si_hbm, di_hbm, eout_hbm, part_hbm,
             idx_s, idx_d, ps, pd, en, agg_sh, sem_g1, sem_g2, sem_st, sem_sc):
    c = lax.axis_index("c")
    s = lax.axis_index("s")
    wid = c * _NS + s

    zero = jnp.zeros((_DE,), jnp.float32)

    def _z(i, carry):
        ps[i] = zero
        return carry

    lax.fori_loop(0, _RPS, _z, 0)
    a0 = pl.multiple_of(s * _RPS, 8)
    pltpu.sync_copy(ps.at[pl.ds(0, _RPS)], agg_sh.at[pl.ds(a0, _RPS)])

    @pl.when(s == 0)
    def _zero_tail():
        pltpu.sync_copy(ps.at[pl.ds(0, _RTL)],
                        agg_sh.at[pl.ds(_RPS * _NS, _RTL)])

    plsc.subcore_barrier()

    def _chunk(g, carry):
        e0 = pl.multiple_of(wid * _EPW + g * _C, _C)
        r0 = pl.multiple_of(wid * (_EPW // _SUB) + g * _NSUB, _NSUB)
        pltpu.sync_copy(si_hbm.at[pl.ds(r0, _NSUB)], idx_s)
        pltpu.sync_copy(di_hbm.at[pl.ds(r0, _NSUB)], idx_d)
        h_base = pltpu.async_copy(base_hbm.at[pl.ds(e0, _C)], en, sem_st)

        def _fire(j, cc):
            pltpu.async_copy(p1_hbm.at[idx_s.at[j]],
                             ps.at[pl.ds(j * _SUB, _SUB)], sem_g1)
            pltpu.async_copy(p2_hbm.at[idx_d.at[j]],
                             pd.at[pl.ds(j * _SUB, _SUB)], sem_g2)
            return cc

        lax.fori_loop(0, _NSUB, _fire, 0)
        h_base.wait()
        pltpu.make_async_copy(eout_hbm.at[pl.ds(0, _C)], ps, sem_g1).wait()
        pltpu.make_async_copy(eout_hbm.at[pl.ds(0, _C)], pd, sem_g2).wait()

        def _cmp(i, cc):
            en[i] = jnp.maximum(en[i] + ps[i] + pd[i], 0.0)
            return cc

        lax.fori_loop(0, _C, _cmp, 0)

        h_out = pltpu.async_copy(en, eout_hbm.at[pl.ds(e0, _C)], sem_st)

        def _scat(j, cc):
            pltpu.async_copy(en.at[pl.ds(j * _SUB, _SUB)],
                             agg_sh.at[idx_d.at[j]], sem_sc, add=True)
            return cc

        lax.fori_loop(0, _NSUB, _scat, 0)
        h_out.wait()
        pltpu.make_async_copy(eout_hbm.at[pl.ds(0, _C)], en, sem_sc).wait()
        return carry

    lax.fori_loop(0, _NCH, _chunk, 0)

    plsc.subcore_barrier()
    pltpu.sync_copy(agg_sh.at[pl.ds(a0, _RPS)], ps.at[pl.ds(0, _RPS)])
    pltpu.sync_copy(ps.at[pl.ds(0, _RPS)],
                    part_hbm.at[c, pl.ds(a0, _RPS)])

    @pl.when(s == 0)
    def _dump_tail():
        pltpu.sync_copy(agg_sh.at[pl.ds(_RPS * _NS, _RTL)],
                        pd.at[pl.ds(0, _RTL)])
        pltpu.sync_copy(pd.at[pl.ds(0, _RTL)],
                        part_hbm.at[c, pl.ds(_RPS * _NS, _RTL)])


@functools.cache
def _sc_edges():
    return functools.partial(
        pl.kernel,
        out_type=(
            jax.ShapeDtypeStruct((_E, _DE), jnp.float32),
            jax.ShapeDtypeStruct((_NC, _N, _DE), jnp.float32),
        ),
        mesh=plsc.VectorSubcoreMesh(core_axis_name="c", subcore_axis_name="s",
                                    num_cores=_NC, num_subcores=_NS),
        scratch_types=[
            pltpu.VMEM((_NSUB, _SUB), jnp.int32),
            pltpu.VMEM((_NSUB, _SUB), jnp.int32),
            pltpu.VMEM((_C, _DE), jnp.float32),
            pltpu.VMEM((_C, _DE), jnp.float32),
            pltpu.VMEM((_C, _DE), jnp.float32),
            pltpu.VMEM_SHARED((_N, _DE), jnp.float32),
            pltpu.SemaphoreType.DMA,
            pltpu.SemaphoreType.DMA,
            pltpu.SemaphoreType.DMA,
            pltpu.SemaphoreType.DMA,
        ],
        compiler_params=pltpu.CompilerParams(use_tc_tiling_on_sc=False),
    )(_sc_body)


def _tc_post_body(x_ref, pp_ref, wn_ref, bn_ref, o_ref):
    x = x_ref[...]
    agg = pp_ref[0] + pp_ref[1]
    wn = wn_ref[...]
    o = (
        jnp.dot(x, wn[0:_D, :], preferred_element_type=jnp.float32)
        + jnp.dot(agg, wn[_D:_D + _DE, :], preferred_element_type=jnp.float32)
        + bn_ref[...]
    )
    o_ref[...] = jnp.maximum(o, 0.0)


_tc_post = pl.pallas_call(
    _tc_post_body,
    out_shape=jax.ShapeDtypeStruct((_N, _D), jnp.float32),
)


def kernel(x, edge_index, edge_attr, W_e, b_e, W_n, b_n):
    src = edge_index[0].reshape(_E // _SUB, _SUB)
    dst = edge_index[1].reshape(_E // _SUB, _SUB)
    p1, p2 = _tc_pre(x, W_e)
    w3b = jnp.kron(jnp.eye(8, dtype=jnp.float32), W_e[2 * _D:])
    b8 = jnp.tile(b_e, 8).reshape(1, 8 * _DE)
    base = _tc_base(edge_attr.reshape(_E8, 8 * _DE), w3b, b8).reshape(_E, _DE)
    eout, part = _sc_edges()(p1, p2, base, src, dst)
    x_new = _tc_post(x, part, W_n, b_n.reshape(1, _D))
    return x_new, eout


# speedup vs baseline: 7.0943x; 1.1411x over previous
import functools

import jax
import jax.numpy as jnp
from jax import lax
from jax.experimental import pallas as pl
from jax.experimental.pallas import tpu as pltpu
from jax.experimental.pallas import tpu_sc as plsc

_N = 10000
_E = 320000
_D = 128
_DE = 16

_NC = 2
_NS = 16
_NW = _NC * _NS
_EPW = _E // _NW
_SUB = 125
_C = 1000
_NSUB = _C // _SUB
_NCH = _EPW // _C
_RPS = 624
_RTL = _N - _RPS * _NS

_E8 = _E // 8
_FW = 1248
_FCP = 312
_FNCH = _FW // _FCP
_FT = _E8 - _FW * _NW


def _tc_pre_body(x_ref, we_ref, p1_ref, p2_ref):
    x = x_ref[...]
    w = we_ref[...]
    p1_ref[...] = jnp.dot(x, w[0:_D, :], preferred_element_type=jnp.float32)
    p2_ref[...] = jnp.dot(x, w[_D:2 * _D, :], preferred_element_type=jnp.float32)


_tc_pre = pl.pallas_call(
    _tc_pre_body,
    out_shape=(
        jax.ShapeDtypeStruct((_N, _DE), jnp.float32),
        jax.ShapeDtypeStruct((_N, _DE), jnp.float32),
    ),
)


def _tc_base_body(ea_ref, w_ref, b_ref, o_ref):
    o_ref[...] = (
        jnp.dot(ea_ref[...], w_ref[...], preferred_element_type=jnp.float32)
        + b_ref[...]
    )


_BLK = 4000
_tc_base = pl.pallas_call(
    _tc_base_body,
    grid=(_E8 // _BLK,),
    in_specs=[
        pl.BlockSpec((_BLK, 8 * _DE), lambda i: (i, 0)),
        pl.BlockSpec((8 * _DE, 8 * _DE), lambda i: (0, 0)),
        pl.BlockSpec((1, 8 * _DE), lambda i: (0, 0)),
    ],
    out_specs=pl.BlockSpec((_BLK, 8 * _DE), lambda i: (i, 0)),
    out_shape=jax.ShapeDtypeStruct((_E8, 8 * _DE), jnp.float32),
)


def _sc_body(p1_hbm, p2_hbm, base_hbm, si_hbm, di_hbm, eout_hbm, part_hbm,
             idx_sA, idx_dA, psA, pdA, bbA,
             idx_sB, idx_dB, psB, pdB, bbB,
             en, agg_sh,
             sg1A, sg2A, sstA, sg1B, sg2B, sstB, sem_st, sem_sc):
    c = lax.axis_index("c")
    s = lax.axis_index("s")
    wid = c * _NS + s

    zero = jnp.zeros((_DE,), jnp.float32)

    def _z(i, carry):
        psA[i] = zero
        return carry

    lax.fori_loop(0, _RPS, _z, 0)
    a0 = pl.multiple_of(s * _RPS, 8)
    pltpu.sync_copy(psA.at[pl.ds(0, _RPS)], agg_sh.at[pl.ds(a0, _RPS)])

    @pl.when(s == 0)
    def _zero_tail():
        pltpu.sync_copy(psA.at[pl.ds(0, _RTL)],
                        agg_sh.at[pl.ds(_RPS * _NS, _RTL)])

    plsc.subcore_barrier()

    def _prefetch(g, idx_s, idx_d, ps, pd, bb, sg1, sg2, sst):
        e0 = pl.multiple_of(wid * _EPW + g * _C, _C)
        r0 = pl.multiple_of(wid * (_EPW // _SUB) + g * _NSUB, _NSUB)
        pltpu.sync_copy(si_hbm.at[pl.ds(r0, _NSUB)], idx_s)
        pltpu.sync_copy(di_hbm.at[pl.ds(r0, _NSUB)], idx_d)
        b0 = pl.multiple_of(e0 * _DE, _C * _DE)
        pltpu.async_copy(base_hbm.at[pl.ds(b0, _C * _DE)], bb, sst)

        def _fire(j, cc):
            pltpu.async_copy(p1_hbm.at[idx_s.at[j]],
                             ps.at[pl.ds(j * _SUB, _SUB)], sg1)
            pltpu.async_copy(p2_hbm.at[idx_d.at[j]],
                             pd.at[pl.ds(j * _SUB, _SUB)], sg2)
            return cc

        lax.fori_loop(0, _NSUB, _fire, 0)

    def _process(g, idx_d, ps, pd, bb, sg1, sg2, sst):
        e0 = pl.multiple_of(wid * _EPW + g * _C, _C)
        pltpu.make_async_copy(base_hbm.at[pl.ds(0, _C * _DE)], bb, sst).wait()
        pltpu.make_async_copy(eout_hbm.at[pl.ds(0, _C)], ps, sg1).wait()
        pltpu.make_async_copy(eout_hbm.at[pl.ds(0, _C)], pd, sg2).wait()

        def _cmp(i, cc):
            en[i] = jnp.maximum(bb[pl.ds(i * _DE, _DE)] + ps[i] + pd[i], 0.0)
            return cc

        lax.fori_loop(0, _C, _cmp, 0)

        h_out = pltpu.async_copy(en, eout_hbm.at[pl.ds(e0, _C)], sem_st)

        def _scat(j, cc):
            pltpu.async_copy(en.at[pl.ds(j * _SUB, _SUB)],
                             agg_sh.at[idx_d.at[j]], sem_sc, add=True)
            return cc

        lax.fori_loop(0, _NSUB, _scat, 0)
        h_out.wait()
        pltpu.make_async_copy(eout_hbm.at[pl.ds(0, _C)], en, sem_sc).wait()

    bufA = (idx_sA, idx_dA, psA, pdA, bbA, sg1A, sg2A, sstA)
    bufB = (idx_sB, idx_dB, psB, pdB, bbB, sg1B, sg2B, sstB)

    _prefetch(0, *bufA)

    def _pair(gg, carry):
        g = gg * 2
        _prefetch(g + 1, *bufB)
        _process(g, *bufA[1:])

        @pl.when(g + 2 < _NCH)
        def _pf_next():
            _prefetch(g + 2, *bufA)

        _process(g + 1, *bufB[1:])
        return carry

    lax.fori_loop(0, _NCH // 2, _pair, 0)

    plsc.subcore_barrier()
    pltpu.sync_copy(agg_sh.at[pl.ds(a0, _RPS)], psA.at[pl.ds(0, _RPS)])
    pltpu.sync_copy(psA.at[pl.ds(0, _RPS)],
                    part_hbm.at[c, pl.ds(a0, _RPS)])

    @pl.when(s == 0)
    def _dump_tail():
        pltpu.sync_copy(agg_sh.at[pl.ds(_RPS * _NS, _RTL)],
                        pdA.at[pl.ds(0, _RTL)])
        pltpu.sync_copy(pdA.at[pl.ds(0, _RTL)],
                        part_hbm.at[c, pl.ds(_RPS * _NS, _RTL)])


@functools.cache
def _sc_edges():
    return functools.partial(
        pl.kernel,
        out_type=(
            jax.ShapeDtypeStruct((_E, _DE), jnp.float32),
            jax.ShapeDtypeStruct((_NC, _N, _DE), jnp.float32),
        ),
        mesh=plsc.VectorSubcoreMesh(core_axis_name="c", subcore_axis_name="s",
                                    num_cores=_NC, num_subcores=_NS),
        scratch_types=[
            pltpu.VMEM((_NSUB, _SUB), jnp.int32),
            pltpu.VMEM((_NSUB, _SUB), jnp.int32),
            pltpu.VMEM((_C, _DE), jnp.float32),
            pltpu.VMEM((_C, _DE), jnp.float32),
            pltpu.VMEM((_C * _DE,), jnp.float32),
            pltpu.VMEM((_NSUB, _SUB), jnp.int32),
            pltpu.VMEM((_NSUB, _SUB), jnp.int32),
            pltpu.VMEM((_C, _DE), jnp.float32),
            pltpu.VMEM((_C, _DE), jnp.float32),
            pltpu.VMEM((_C * _DE,), jnp.float32),
            pltpu.VMEM((_C, _DE), jnp.float32),
            pltpu.VMEM_SHARED((_N, _DE), jnp.float32),
            pltpu.SemaphoreType.DMA,
            pltpu.SemaphoreType.DMA,
            pltpu.SemaphoreType.DMA,
            pltpu.SemaphoreType.DMA,
            pltpu.SemaphoreType.DMA,
            pltpu.SemaphoreType.DMA,
            pltpu.SemaphoreType.DMA,
            pltpu.SemaphoreType.DMA,
        ],
        compiler_params=pltpu.CompilerParams(use_tc_tiling_on_sc=False),
    )(_sc_body)


def _tc_post_body(x_ref, pp_ref, wn_ref, bn_ref, o_ref):
    x = x_ref[...]
    agg = pp_ref[0] + pp_ref[1]
    wn = wn_ref[...]
    o = (
        jnp.dot(x, wn[0:_D, :], preferred_element_type=jnp.float32)
        + jnp.dot(agg, wn[_D:_D + _DE, :], preferred_element_type=jnp.float32)
        + bn_ref[...]
    )
    o_ref[...] = jnp.maximum(o, 0.0)


_tc_post = pl.pallas_call(
    _tc_post_body,
    out_shape=jax.ShapeDtypeStruct((_N, _D), jnp.float32),
)


def kernel(x, edge_index, edge_attr, W_e, b_e, W_n, b_n):
    src = edge_index[0].reshape(_E // _SUB, _SUB)
    dst = edge_index[1].reshape(_E // _SUB, _SUB)
    p1, p2 = _tc_pre(x, W_e)
    w3b = jnp.kron(jnp.eye(8, dtype=jnp.float32), W_e[2 * _D:])
    b8 = jnp.tile(b_e, 8).reshape(1, 8 * _DE)
    base = _tc_base(edge_attr.reshape(_E8, 8 * _DE), w3b, b8).reshape(-1)
    eout, part = _sc_edges()(p1, p2, base, src, dst)
    x_new = _tc_post(x, part, W_n, b_n.reshape(1, _D))
    return x_new, eout


# speedup vs baseline: 9.5732x; 1.3494x over previous
import functools

import jax
import jax.numpy as jnp
from jax import lax
from jax.experimental import pallas as pl
from jax.experimental.pallas import tpu as pltpu
from jax.experimental.pallas import tpu_sc as plsc

_N = 10000
_E = 320000
_D = 128
_DE = 16

_NC = 2
_NS = 16
_NW = _NC * _NS
_EPW = _E // _NW
_SUB = 125
_C = 1000
_NSUB = _C // _SUB
_NCH = _EPW // _C
_RPS = 624
_RTL = _N - _RPS * _NS

_E8 = _E // 8
_FW = 1248
_FCP = 312
_FNCH = _FW // _FCP
_FT = _E8 - _FW * _NW


def _tc_pre_body(x_ref, we_ref, p1_ref, p2_ref):
    x = x_ref[...]
    w = we_ref[...]
    p1_ref[...] = jnp.dot(x, w[0:_D, :], preferred_element_type=jnp.float32)
    p2_ref[...] = jnp.dot(x, w[_D:2 * _D, :], preferred_element_type=jnp.float32)


_tc_pre = pl.pallas_call(
    _tc_pre_body,
    out_shape=(
        jax.ShapeDtypeStruct((_N, _DE), jnp.float32),
        jax.ShapeDtypeStruct((_N, _DE), jnp.float32),
    ),
)


def _tc_base_body(ea_ref, w_ref, b_ref, o_ref):
    o_ref[...] = (
        jnp.dot(ea_ref[...], w_ref[...], preferred_element_type=jnp.float32)
        + b_ref[...]
    )


_BLK = 4000
_tc_base = pl.pallas_call(
    _tc_base_body,
    grid=(_E8 // _BLK,),
    in_specs=[
        pl.BlockSpec((_BLK, 8 * _DE), lambda i: (i, 0)),
        pl.BlockSpec((8 * _DE, 8 * _DE), lambda i: (0, 0)),
        pl.BlockSpec((1, 8 * _DE), lambda i: (0, 0)),
    ],
    out_specs=pl.BlockSpec((_BLK, 8 * _DE), lambda i: (i, 0)),
    out_shape=jax.ShapeDtypeStruct((_E8, 8 * _DE), jnp.float32),
)


def _sc_body(p1_hbm, p2_hbm, base_hbm, si_hbm, di_hbm, eout_hbm, part_hbm,
             idx_sA, idx_dA, psA, pdA, bbA,
             idx_sB, idx_dB, psB, pdB, bbB,
             en, agg_sh,
             sg1A, sg2A, sstA, sg1B, sg2B, sstB, sem_st, sem_sc):
    c = lax.axis_index("c")
    s = lax.axis_index("s")
    wid = c * _NS + s

    zero = jnp.zeros((_DE,), jnp.float32)

    def _z(i, carry):
        psA[i] = zero
        return carry

    lax.fori_loop(0, _RPS, _z, 0)
    a0 = pl.multiple_of(s * _RPS, 8)
    pltpu.sync_copy(psA.at[pl.ds(0, _RPS)], agg_sh.at[pl.ds(a0, _RPS)])

    @pl.when(s == 0)
    def _zero_tail():
        pltpu.sync_copy(psA.at[pl.ds(0, _RTL)],
                        agg_sh.at[pl.ds(_RPS * _NS, _RTL)])

    plsc.subcore_barrier()

    def _prefetch(g, idx_s, idx_d, ps, pd, bb, sg1, sg2, sst):
        e0 = pl.multiple_of(wid * _EPW + g * _C, _C)
        r0 = pl.multiple_of(wid * (_EPW // _SUB) + g * _NSUB, _NSUB)
        pltpu.sync_copy(si_hbm.at[pl.ds(r0, _NSUB)], idx_s)
        pltpu.sync_copy(di_hbm.at[pl.ds(r0, _NSUB)], idx_d)
        b0 = pl.multiple_of(e0 * _DE, _C * _DE)
        pltpu.async_copy(base_hbm.at[pl.ds(b0, _C * _DE)], bb, sst)

        def _fire(j, cc):
            pltpu.async_copy(p1_hbm.at[idx_s.at[j]],
                             ps.at[pl.ds(j * _SUB, _SUB)], sg1)
            pltpu.async_copy(p2_hbm.at[idx_d.at[j]],
                             pd.at[pl.ds(j * _SUB, _SUB)], sg2)
            return cc

        lax.fori_loop(0, _NSUB, _fire, 0)

    def _process(g, idx_d, ps, pd, bb, sg1, sg2, sst):
        e0 = pl.multiple_of(wid * _EPW + g * _C, _C)
        pltpu.make_async_copy(base_hbm.at[pl.ds(0, _C * _DE)], bb, sst).wait()
        pltpu.make_async_copy(
            eout_hbm.at[pl.ds(0, _C), pl.ds(0, _DE)], ps, sg1).wait()
        pltpu.make_async_copy(
            eout_hbm.at[pl.ds(0, _C), pl.ds(0, _DE)], pd, sg2).wait()

        def _cmp(i, cc):
            en[i] = jnp.maximum(bb[pl.ds(i * _DE, _DE)] + ps[i] + pd[i], 0.0)
            return cc

        lax.fori_loop(0, _C, _cmp, 0)

        h_out = pltpu.async_copy(
            en, eout_hbm.at[pl.ds(e0, _C), pl.ds(0, _DE)], sem_st)

        def _scat(j, cc):
            pltpu.async_copy(en.at[pl.ds(j * _SUB, _SUB)],
                             agg_sh.at[idx_d.at[j]], sem_sc, add=True)
            return cc

        lax.fori_loop(0, _NSUB, _scat, 0)
        h_out.wait()
        pltpu.make_async_copy(
            eout_hbm.at[pl.ds(0, _C), pl.ds(0, _DE)], en, sem_sc).wait()

    bufA = (idx_sA, idx_dA, psA, pdA, bbA, sg1A, sg2A, sstA)
    bufB = (idx_sB, idx_dB, psB, pdB, bbB, sg1B, sg2B, sstB)

    _prefetch(0, *bufA)

    def _pair(gg, carry):
        g = gg * 2
        _prefetch(g + 1, *bufB)
        _process(g, *bufA[1:])

        @pl.when(g + 2 < _NCH)
        def _pf_next():
            _prefetch(g + 2, *bufA)

        _process(g + 1, *bufB[1:])
        return carry

    lax.fori_loop(0, _NCH // 2, _pair, 0)

    plsc.subcore_barrier()
    pltpu.sync_copy(agg_sh.at[pl.ds(a0, _RPS)], psA.at[pl.ds(0, _RPS)])
    pltpu.sync_copy(psA.at[pl.ds(0, _RPS)],
                    part_hbm.at[c, pl.ds(a0, _RPS)])

    @pl.when(s == 0)
    def _dump_tail():
        pltpu.sync_copy(agg_sh.at[pl.ds(_RPS * _NS, _RTL)],
                        pdA.at[pl.ds(0, _RTL)])
        pltpu.sync_copy(pdA.at[pl.ds(0, _RTL)],
                        part_hbm.at[c, pl.ds(_RPS * _NS, _RTL)])


@functools.cache
def _sc_edges():
    return functools.partial(
        pl.kernel,
        out_type=(
            jax.ShapeDtypeStruct((_E, 8 * _DE), jnp.float32),
            jax.ShapeDtypeStruct((_NC, _N, _DE), jnp.float32),
        ),
        mesh=plsc.VectorSubcoreMesh(core_axis_name="c", subcore_axis_name="s",
                                    num_cores=_NC, num_subcores=_NS),
        scratch_types=[
            pltpu.VMEM((_NSUB, _SUB), jnp.int32),
            pltpu.VMEM((_NSUB, _SUB), jnp.int32),
            pltpu.VMEM((_C, _DE), jnp.float32),
            pltpu.VMEM((_C, _DE), jnp.float32),
            pltpu.VMEM((_C * _DE,), jnp.float32),
            pltpu.VMEM((_NSUB, _SUB), jnp.int32),
            pltpu.VMEM((_NSUB, _SUB), jnp.int32),
            pltpu.VMEM((_C, _DE), jnp.float32),
            pltpu.VMEM((_C, _DE), jnp.float32),
            pltpu.VMEM((_C * _DE,), jnp.float32),
            pltpu.VMEM((_C, _DE), jnp.float32),
            pltpu.VMEM_SHARED((_N, _DE), jnp.float32),
            pltpu.SemaphoreType.DMA,
            pltpu.SemaphoreType.DMA,
            pltpu.SemaphoreType.DMA,
            pltpu.SemaphoreType.DMA,
            pltpu.SemaphoreType.DMA,
            pltpu.SemaphoreType.DMA,
            pltpu.SemaphoreType.DMA,
            pltpu.SemaphoreType.DMA,
        ],
        compiler_params=pltpu.CompilerParams(use_tc_tiling_on_sc=False),
    )(_sc_body)


def _tc_post_body(x_ref, pp_ref, wn_ref, bn_ref, o_ref):
    x = x_ref[...]
    agg = pp_ref[0] + pp_ref[1]
    wn = wn_ref[...]
    o = (
        jnp.dot(x, wn[0:_D, :], preferred_element_type=jnp.float32)
        + jnp.dot(agg, wn[_D:_D + _DE, :], preferred_element_type=jnp.float32)
        + bn_ref[...]
    )
    o_ref[...] = jnp.maximum(o, 0.0)


_tc_post = pl.pallas_call(
    _tc_post_body,
    out_shape=jax.ShapeDtypeStruct((_N, _D), jnp.float32),
)


def kernel(x, edge_index, edge_attr, W_e, b_e, W_n, b_n):
    src = edge_index[0].reshape(_E // _SUB, _SUB)
    dst = edge_index[1].reshape(_E // _SUB, _SUB)
    p1, p2 = _tc_pre(x, W_e)
    w3b = jnp.kron(jnp.eye(8, dtype=jnp.float32), W_e[2 * _D:])
    b8 = jnp.tile(b_e, 8).reshape(1, 8 * _DE)
    base = _tc_base(edge_attr.reshape(_E8, 8 * _DE), w3b, b8).reshape(-1)
    eout128, part = _sc_edges()(p1, p2, base, src, dst)
    x_new = _tc_post(x, part, W_n, b_n.reshape(1, _D))
    return x_new, eout128[:, :_DE]


# speedup vs baseline: 9.6299x; 1.0059x over previous
import functools

import jax
import jax.numpy as jnp
from jax import lax
from jax.experimental import pallas as pl
from jax.experimental.pallas import tpu as pltpu
from jax.experimental.pallas import tpu_sc as plsc

_N = 10000
_E = 320000
_D = 128
_DE = 16

_NC = 2
_NS = 16
_NW = _NC * _NS
_EPW = _E // _NW
_SUB = 125
_C = 1000
_NSUB = _C // _SUB
_NCH = _EPW // _C
_RPS = 624
_RTL = _N - _RPS * _NS

_E8 = _E // 8
_FW = 1248
_FCP = 312
_FNCH = _FW // _FCP
_FT = _E8 - _FW * _NW


def _tc_pre_body(x_ref, we_ref, p1_ref, p2_ref):
    x = x_ref[...]
    w = we_ref[...]
    p1_ref[...] = jnp.dot(x, w[0:_D, :], preferred_element_type=jnp.float32)
    p2_ref[...] = jnp.dot(x, w[_D:2 * _D, :], preferred_element_type=jnp.float32)


_tc_pre = pl.pallas_call(
    _tc_pre_body,
    out_shape=(
        jax.ShapeDtypeStruct((_N, _DE), jnp.float32),
        jax.ShapeDtypeStruct((_N, _DE), jnp.float32),
    ),
)


def _tc_base_body(ea_ref, w_ref, b_ref, o_ref):
    o_ref[...] = (
        jnp.dot(ea_ref[...], w_ref[...], preferred_element_type=jnp.float32)
        + b_ref[...]
    )


_BLK = 4000
_tc_base = pl.pallas_call(
    _tc_base_body,
    grid=(_E8 // _BLK,),
    in_specs=[
        pl.BlockSpec((_BLK, 8 * _DE), lambda i: (i, 0)),
        pl.BlockSpec((8 * _DE, 8 * _DE), lambda i: (0, 0)),
        pl.BlockSpec((1, 8 * _DE), lambda i: (0, 0)),
    ],
    out_specs=pl.BlockSpec((_BLK, 8 * _DE), lambda i: (i, 0)),
    out_shape=jax.ShapeDtypeStruct((_E8, 8 * _DE), jnp.float32),
)


def _sc_body(p1_hbm, p2_hbm, base_hbm, si_hbm, di_hbm, eout_hbm, part_hbm,
             idx_sA, idx_dA, psA, pdA, bbA,
             idx_sB, idx_dB, psB, pdB, bbB,
             en, agg_sh,
             sg1A, sg2A, sstA, sg1B, sg2B, sstB, sem_st, sem_sc):
    c = lax.axis_index("c")
    s = lax.axis_index("s")
    wid = c * _NS + s

    zero = jnp.zeros((_DE,), jnp.float32)

    def _z(i, carry):
        psA[i] = zero
        return carry

    lax.fori_loop(0, _RPS, _z, 0)
    a0 = pl.multiple_of(s * _RPS, 8)
    pltpu.sync_copy(psA.at[pl.ds(0, _RPS)], agg_sh.at[pl.ds(a0, _RPS)])

    @pl.when(s == 0)
    def _zero_tail():
        pltpu.sync_copy(psA.at[pl.ds(0, _RTL)],
                        agg_sh.at[pl.ds(_RPS * _NS, _RTL)])

    plsc.subcore_barrier()

    def _prefetch(g, idx_s, idx_d, ps, pd, bb, sg1, sg2, sst):
        e0 = pl.multiple_of(wid * _EPW + g * _C, _C)
        r0 = pl.multiple_of(wid * (_EPW // _SUB) + g * _NSUB, _NSUB)
        pltpu.sync_copy(si_hbm.at[pl.ds(r0, _NSUB)], idx_s)
        pltpu.sync_copy(di_hbm.at[pl.ds(r0, _NSUB)], idx_d)
        b0 = pl.multiple_of(e0 * _DE, _C * _DE)
        pltpu.async_copy(base_hbm.at[pl.ds(b0, _C * _DE)], bb, sst)

        def _fire(j, cc):
            pltpu.async_copy(p1_hbm.at[idx_s.at[j]],
                             ps.at[pl.ds(j * _SUB, _SUB)], sg1)
            pltpu.async_copy(p2_hbm.at[idx_d.at[j]],
                             pd.at[pl.ds(j * _SUB, _SUB)], sg2)
            return cc

        lax.fori_loop(0, _NSUB, _fire, 0)

    def _process(g, idx_d, ps, pd, bb, sg1, sg2, sst):
        e0 = pl.multiple_of(wid * _EPW + g * _C, _C)
        pltpu.make_async_copy(base_hbm.at[pl.ds(0, _C * _DE)], bb, sst).wait()
        pltpu.make_async_copy(
            eout_hbm.at[pl.ds(0, _C), pl.ds(0, _DE)], ps, sg1).wait()
        pltpu.make_async_copy(
            eout_hbm.at[pl.ds(0, _C), pl.ds(0, _DE)], pd, sg2).wait()

        def _cmp(i, cc):
            en[i] = jnp.maximum(bb[pl.ds(i * _DE, _DE)] + ps[i] + pd[i], 0.0)
            return cc

        lax.fori_loop(0, _C, _cmp, 0)

        h_out = pltpu.async_copy(
            en, eout_hbm.at[pl.ds(e0, _C), pl.ds(0, _DE)], sem_st)

        def _scat(j, cc):
            pltpu.async_copy(en.at[pl.ds(j * _SUB, _SUB)],
                             agg_sh.at[idx_d.at[j]], sem_sc, add=True)
            return cc

        lax.fori_loop(0, _NSUB, _scat, 0)
        h_out.wait()
        pltpu.make_async_copy(
            eout_hbm.at[pl.ds(0, _C), pl.ds(0, _DE)], en, sem_sc).wait()

    bufA = (idx_sA, idx_dA, psA, pdA, bbA, sg1A, sg2A, sstA)
    bufB = (idx_sB, idx_dB, psB, pdB, bbB, sg1B, sg2B, sstB)

    _prefetch(0, *bufA)

    def _pair(gg, carry):
        g = gg * 2
        _prefetch(g + 1, *bufB)
        _process(g, *bufA[1:])

        @pl.when(g + 2 < _NCH)
        def _pf_next():
            _prefetch(g + 2, *bufA)

        _process(g + 1, *bufB[1:])
        return carry

    lax.fori_loop(0, _NCH // 2, _pair, 0)

    plsc.subcore_barrier()
    pltpu.sync_copy(agg_sh.at[pl.ds(a0, _RPS)], psA.at[pl.ds(0, _RPS)])
    pltpu.sync_copy(psA.at[pl.ds(0, _RPS)],
                    part_hbm.at[c, pl.ds(a0, _RPS), pl.ds(0, _DE)])

    @pl.when(s == 0)
    def _dump_tail():
        pltpu.sync_copy(agg_sh.at[pl.ds(_RPS * _NS, _RTL)],
                        pdA.at[pl.ds(0, _RTL)])
        pltpu.sync_copy(pdA.at[pl.ds(0, _RTL)],
                        part_hbm.at[c, pl.ds(_RPS * _NS, _RTL), pl.ds(0, _DE)])


@functools.cache
def _sc_edges():
    return functools.partial(
        pl.kernel,
        out_type=(
            jax.ShapeDtypeStruct((_E, 8 * _DE), jnp.float32),
            jax.ShapeDtypeStruct((_NC, _N, 8 * _DE), jnp.float32),
        ),
        mesh=plsc.VectorSubcoreMesh(core_axis_name="c", subcore_axis_name="s",
                                    num_cores=_NC, num_subcores=_NS),
        scratch_types=[
            pltpu.VMEM((_NSUB, _SUB), jnp.int32),
            pltpu.VMEM((_NSUB, _SUB), jnp.int32),
            pltpu.VMEM((_C, _DE), jnp.float32),
            pltpu.VMEM((_C, _DE), jnp.float32),
            pltpu.VMEM((_C * _DE,), jnp.float32),
            pltpu.VMEM((_NSUB, _SUB), jnp.int32),
            pltpu.VMEM((_NSUB, _SUB), jnp.int32),
            pltpu.VMEM((_C, _DE), jnp.float32),
            pltpu.VMEM((_C, _DE), jnp.float32),
            pltpu.VMEM((_C * _DE,), jnp.float32),
            pltpu.VMEM((_C, _DE), jnp.float32),
            pltpu.VMEM_SHARED((_N, _DE), jnp.float32),
            pltpu.SemaphoreType.DMA,
            pltpu.SemaphoreType.DMA,
            pltpu.SemaphoreType.DMA,
            pltpu.SemaphoreType.DMA,
            pltpu.SemaphoreType.DMA,
            pltpu.SemaphoreType.DMA,
            pltpu.SemaphoreType.DMA,
            pltpu.SemaphoreType.DMA,
        ],
        compiler_params=pltpu.CompilerParams(use_tc_tiling_on_sc=False),
    )(_sc_body)


def _tc_post_body(x_ref, pp_ref, wn_ref, bn_ref, o_ref):
    x = x_ref[...]
    agg = pp_ref[0] + pp_ref[1]
    wn = wn_ref[...]
    o = (
        jnp.dot(x, wn[0:_D, :], preferred_element_type=jnp.float32)
        + jnp.dot(agg, wn[_D:_D + _DE, :], preferred_element_type=jnp.float32)
        + bn_ref[...]
    )
    o_ref[...] = jnp.maximum(o, 0.0)


_tc_post = pl.pallas_call(
    _tc_post_body,
    out_shape=jax.ShapeDtypeStruct((_N, _D), jnp.float32),
)


def kernel(x, edge_index, edge_attr, W_e, b_e, W_n, b_n):
    src = edge_index[0].reshape(_E // _SUB, _SUB)
    dst = edge_index[1].reshape(_E // _SUB, _SUB)
    p1, p2 = _tc_pre(x, W_e)
    w3b = jnp.kron(jnp.eye(8, dtype=jnp.float32), W_e[2 * _D:])
    b8 = jnp.tile(b_e, 8).reshape(1, 8 * _DE)
    base = _tc_base(edge_attr.reshape(_E8, 8 * _DE), w3b, b8).reshape(-1)
    eout128, part128 = _sc_edges()(p1, p2, base, src, dst)
    x_new = _tc_post(x, part128[:, :, :_DE], W_n, b_n.reshape(1, _D))
    return x_new, eout128[:, :_DE]
